# Initial kernel scaffold; baseline (speedup 1.0000x reference)
import jax
import jax.numpy as jnp
from jax.experimental import pallas as pl


def kernel(batch_node_features, batch_edge_indices, W1, a_src1, a_dst1, b1, W2, a_src2, a_dst2, b2, W3, a_src3, a_dst3, b3, W4, a_src4, a_dst4, b4, fc1_W, fc1_b, fc2_W, fc2_b, fc3_W, fc3_b):
    raise NotImplementedError("write your pallas kernel here")


# speedup vs baseline: 14.6196x; 14.6196x over previous
import functools

import jax
import jax.numpy as jnp
from jax import lax
from jax.experimental import pallas as pl
from jax.experimental.pallas import tpu as pltpu
from jax.experimental.pallas import tpu_sc as plsc

_N = 10000
_H = 152
_HP = 160
_HC = 80
_E = 320000
_ETOT = _E + _N
_K = 128
_NSUB = 162
_CHUNK = _NSUB * _K
_EPAD = 16 * _CHUNK
_NP = 10240
_RPS = _NP // 16
_BLK = 2000
_BLKH = 400


def _split_h(hp, h_ref, as_ref, ad_ref):
    as_ref[0] = hp[:, _H + 1:_H + 2]
    ad_ref[0] = hp[:, _H + 2:_H + 3]
    h_ref[0, 0] = hp[:, :_HC]
    colb = lax.broadcasted_iota(jnp.int32, (_BLK, _HC), 1)
    h_ref[1, 0] = hp[:, _HC:_HP] + (colb == (_H - _HC)).astype(jnp.float32)


def _h_body(x_ref, w_ref, h_ref, as_ref, ad_ref):
    hp = jnp.dot(x_ref[0], w_ref[...], preferred_element_type=jnp.float32)
    _split_h(hp, h_ref, as_ref, ad_ref)


def _h_out_specs(bb):
    return dict(
        out_specs=[
            pl.BlockSpec((2, 1, _BLK, _HC), lambda b, i: (0, b, i, 0)),
            pl.BlockSpec((1, _BLK, 1), lambda b, i: (b, i, 0)),
            pl.BlockSpec((1, _BLK, 1), lambda b, i: (b, i, 0)),
        ],
        out_shape=[
            jax.ShapeDtypeStruct((2, bb, _N, _HC), jnp.float32),
            jax.ShapeDtypeStruct((bb, _N, 1), jnp.float32),
            jax.ShapeDtypeStruct((bb, _N, 1), jnp.float32),
        ],
    )


def _h_call(x, w_aug):
    bb, n, din = x.shape
    return pl.pallas_call(
        _h_body,
        grid=(bb, n // _BLK),
        in_specs=[
            pl.BlockSpec((1, _BLK, din), lambda b, i: (b, i, 0)),
            pl.BlockSpec((din, _HP), lambda b, i: (0, 0)),
        ],
        **_h_out_specs(bb),
    )(x, w_aug)


def _combine(p_ref, b_ref):
    sall = jnp.concatenate([p_ref[0, 0], p_ref[0, 1]], axis=1)
    den = sall[:, _H:_H + 1]
    return sall[:, :_H] / (den + 1e-16) + b_ref[...]


def _hx_body(p_ref, b_ref, w_ref, x_ref, h_ref, as_ref, ad_ref):
    xx = _combine(p_ref, b_ref)
    x_ref[0] = xx
    hp = jnp.dot(xx, w_ref[...], preferred_element_type=jnp.float32)
    _split_h(hp, h_ref, as_ref, ad_ref)


def _hx_call(parts, bvec, w_aug):
    bb = parts.shape[0]
    specs = _h_out_specs(bb)
    return pl.pallas_call(
        _hx_body,
        grid=(bb, _N // _BLK),
        in_specs=[
            pl.BlockSpec((1, 2, _BLK, _HC), lambda b, i: (b, 0, i, 0)),
            pl.BlockSpec((1, _H), lambda b, i: (0, 0)),
            pl.BlockSpec((_H, _HP), lambda b, i: (0, 0)),
        ],
        out_specs=[pl.BlockSpec((1, _BLK, _H), lambda b, i: (b, i, 0))]
        + specs["out_specs"],
        out_shape=[jax.ShapeDtypeStruct((bb, _N, _H), jnp.float32)]
        + specs["out_shape"],
    )(parts, bvec, w_aug)


def _fin_body(p_ref, b_ref, x_ref):
    x_ref[0] = _combine(p_ref, b_ref)


def _fin_call(parts, bvec):
    bb = parts.shape[0]
    return pl.pallas_call(
        _fin_body,
        grid=(bb, _N // _BLK),
        in_specs=[
            pl.BlockSpec((1, 2, _BLK, _HC), lambda b, i: (b, 0, i, 0)),
            pl.BlockSpec((1, _H), lambda b, i: (0, 0)),
        ],
        out_specs=[pl.BlockSpec((1, _BLK, _H), lambda b, i: (b, i, 0))],
        out_shape=[jax.ShapeDtypeStruct((bb, _N, _H), jnp.float32)],
    )(parts, bvec)[0]


def _head_body(x0_ref, x1_ref, x2_ref, x3_ref, x4_ref,
               w1_ref, b1_ref, w2_ref, b2_ref, w3_ref, b3_ref,
               o_ref, acc):
    i = pl.program_id(0)

    @pl.when(i == 0)
    def _():
        acc[...] = jnp.zeros_like(acc)

    acc[:, 0:8] += jnp.sum(x0_ref[...], axis=1)
    acc[:, 8:160] += jnp.sum(x1_ref[...], axis=1)
    acc[:, 160:312] += jnp.sum(x2_ref[...], axis=1)
    acc[:, 312:464] += jnp.sum(x3_ref[...], axis=1)
    acc[:, 464:616] += jnp.sum(x4_ref[...], axis=1)

    @pl.when(i == pl.num_programs(0) - 1)
    def _():
        feat = acc[...] * jnp.float32(1.0 / _N)
        f = jnp.dot(feat, w1_ref[...], preferred_element_type=jnp.float32) + b1_ref[...]
        f = jnp.dot(f, w2_ref[...], preferred_element_type=jnp.float32) + b2_ref[...]
        o_ref[...] = jnp.dot(f, w3_ref[...], preferred_element_type=jnp.float32) + b3_ref[...]


def _head_call(x0p, x1, x2, x3, x4, w1p, b1v, w2, b2v, w3, b3v):
    bb = x0p.shape[0]
    return pl.pallas_call(
        _head_body,
        grid=(_N // _BLKH,),
        in_specs=[
            pl.BlockSpec((bb, _BLKH, 8), lambda i: (0, i, 0)),
            pl.BlockSpec((bb, _BLKH, _H), lambda i: (0, i, 0)),
            pl.BlockSpec((bb, _BLKH, _H), lambda i: (0, i, 0)),
            pl.BlockSpec((bb, _BLKH, _H), lambda i: (0, i, 0)),
            pl.BlockSpec((bb, _BLKH, _H), lambda i: (0, i, 0)),
            pl.BlockSpec((616, 256), lambda i: (0, 0)),
            pl.BlockSpec((1, 256), lambda i: (0, 0)),
            pl.BlockSpec((256, 128), lambda i: (0, 0)),
            pl.BlockSpec((1, 128), lambda i: (0, 0)),
            pl.BlockSpec((128, 10), lambda i: (0, 0)),
            pl.BlockSpec((1, 10), lambda i: (0, 0)),
        ],
        out_specs=[pl.BlockSpec((bb, 10), lambda i: (0, 0))],
        out_shape=[jax.ShapeDtypeStruct((bb, 10), jnp.float32)],
        scratch_shapes=[pltpu.VMEM((bb, 616), jnp.float32)],
    )(x0p, x1, x2, x3, x4, w1p, b1v, w2, b2v, w3, b3v)[0]


def _make_agg(bb):
    mesh = plsc.VectorSubcoreMesh(core_axis_name="c", subcore_axis_name="s")

    @functools.partial(
        pl.kernel,
        mesh=mesh,
        compiler_params=pltpu.CompilerParams(
            needs_layout_passes=False, use_tc_tiling_on_sc=False),
        out_type=jax.ShapeDtypeStruct((bb, 2, _NP, _HC), jnp.float32),
        scratch_types=[
            pltpu.VMEM((_N,), jnp.float32),
            pltpu.VMEM((_N,), jnp.float32),
            pltpu.VMEM((_K,), jnp.int32),
            pltpu.VMEM((_K,), jnp.int32),
            pltpu.VMEM((_K, _HC), jnp.float32),
            pltpu.VMEM((_K,), jnp.float32),
            pltpu.VMEM((40, _HC), jnp.float32),
            pltpu.VMEM_SHARED((_NP, _HC), jnp.float32),
        ],
    )
    def agg(h_hbm, as_hbm, ad_hbm, src_hbm, dst_hbm, out_hbm,
            asv, adv, si, di, rows, exb, zbuf, shared):
        c = lax.axis_index("c")
        s = lax.axis_index("s")
        rbase = s * _RPS
        coff = c * (bb * _N)
        zv = jnp.zeros((16,), jnp.float32)
        for i in range(40):
            for j in range(_HC // 16):
                zbuf[i, pl.ds(j * 16, 16)] = zv

        for b in range(bb):
            def _z(t, _):
                pltpu.sync_copy(zbuf, shared.at[pl.ds(rbase + t * 40, 40)])
                return 0
            lax.fori_loop(0, _RPS // 40, _z, 0)
            pltpu.sync_copy(as_hbm.at[pl.ds(b * _N, _N)], asv)
            pltpu.sync_copy(ad_hbm.at[pl.ds(b * _N, _N)], adv)
            plsc.subcore_barrier()

            roff = coff + b * _N
            ebase = b * _EPAD

            def _sub(t, _):
                off = s * _CHUNK + t * _K
                pltpu.sync_copy(src_hbm.at[pl.ds(ebase + off, _K)], si)
                pltpu.sync_copy(dst_hbm.at[pl.ds(ebase + off, _K)], di)

                def _ex(g, _2):
                    sl = pl.ds(g * 16, 16)
                    sv = si[sl]
                    dv = di[sl]
                    av = plsc.load_gather(asv, [sv])
                    bv = plsc.load_gather(adv, [dv])
                    al = av + bv
                    al = jnp.where(al >= 0.0, al, al * 0.2)
                    ex = jnp.exp(al)
                    gi = off + g * 16 + lax.iota(jnp.int32, 16)
                    exb[sl] = jnp.where(gi < _ETOT, ex, 0.0)
                    si[sl] = sv + roff
                    return 0
                lax.fori_loop(0, _K // 16, _ex, 0)

                pltpu.sync_copy(h_hbm.at[si], rows)

                def _sc(e, _2):
                    evec = plsc.load_gather(exb, [jnp.full((16,), e, jnp.int32)])
                    for j in range(_HC // 16):
                        sl2 = pl.ds(j * 16, 16)
                        rows[e, sl2] = rows[e, sl2] * evec
                    return 0
                lax.fori_loop(0, _K, _sc, 0)

                pltpu.sync_copy(rows, shared.at[di], add=True)
                return 0
            lax.fori_loop(0, _NSUB, _sub, 0)
            plsc.subcore_barrier()

            pltpu.sync_copy(shared.at[pl.ds(rbase, _RPS)],
                            out_hbm.at[b, c, pl.ds(rbase, _RPS)])
            plsc.subcore_barrier()

    return agg


def kernel(batch_node_features, batch_edge_indices,
           W1, a_src1, a_dst1, b1, W2, a_src2, a_dst2, b2,
           W3, a_src3, a_dst3, b3, W4, a_src4, a_dst4, b4,
           fc1_W, fc1_b, fc2_W, fc2_b, fc3_W, fc3_b):
    f32 = jnp.float32
    bb, n, cdim = batch_node_features.shape

    ei = batch_edge_indices.astype(jnp.int32)
    arb = jnp.broadcast_to(jnp.arange(_N, dtype=jnp.int32), (bb, _N))
    src = jnp.pad(jnp.concatenate([ei[:, 0, :], arb], axis=1),
                  ((0, 0), (0, _EPAD - _ETOT))).reshape(bb * _EPAD)
    dst = jnp.pad(jnp.concatenate([ei[:, 1, :], arb], axis=1),
                  ((0, 0), (0, _EPAD - _ETOT))).reshape(bb * _EPAD)

    def aug(w, a_s, a_d):
        din = w.shape[0]
        return jnp.concatenate(
            [w, jnp.zeros((din, 1), f32),
             (w @ a_s)[:, None], (w @ a_d)[:, None],
             jnp.zeros((din, _HP - _H - 3), f32)], axis=1)

    wa1 = jnp.pad(aug(W1, a_src1, a_dst1), ((0, 8 - cdim), (0, 0)))
    wa2 = aug(W2, a_src2, a_dst2)
    wa3 = aug(W3, a_src3, a_dst3)
    wa4 = aug(W4, a_src4, a_dst4)
    x0p = jnp.pad(batch_node_features, ((0, 0), (0, 0), (0, 8 - cdim)))

    agg = _make_agg(bb)

    def run_agg(h, asx, adx):
        return agg(h.reshape(2 * bb * _N, _HC), asx.reshape(bb * _N),
                   adx.reshape(bb * _N), src, dst)

    h, asx, adx = _h_call(x0p, wa1)
    parts = run_agg(h, asx, adx)
    x1, h, asx, adx = _hx_call(parts, b1[None, :], wa2)
    parts = run_agg(h, asx, adx)
    x2, h, asx, adx = _hx_call(parts, b2[None, :], wa3)
    parts = run_agg(h, asx, adx)
    x3, h, asx, adx = _hx_call(parts, b3[None, :], wa4)
    parts = run_agg(h, asx, adx)
    x4 = _fin_call(parts, b4[None, :])

    w1p = jnp.concatenate(
        [fc1_W[:cdim], jnp.zeros((8 - cdim, fc1_W.shape[1]), f32), fc1_W[cdim:]],
        axis=0)
    return _head_call(x0p, x1, x2, x3, x4, w1p, fc1_b[None, :],
                      fc2_W, fc2_b[None, :], fc3_W, fc3_b[None, :])
